# baseline (speedup 1.0000x reference)
import functools

import jax
import jax.numpy as jnp
from jax import lax
from jax.experimental import pallas as pl
from jax.experimental.pallas import tpu as pltpu
from jax.experimental.pallas import tpu_sc as plsc

_CROSS_AVG_ERR = 0.25
_CROSS_WEIGHT = 1.0
_MIN_SUPPORT = 20.0
_MAX_BEAR = 20.0

_B = 8
_NP = 2048
_NF = 2048
_KP = 16
_D = 128
_NCHUNK = 4
_QC = _NP // _NCHUNK


def _sc_chamfer_kernel(pts, defo, segp, segf, prm, out,
                       y_v, q_v, sp_v, sf_v, pm_v, res_v):
    c = lax.axis_index("c")
    s = lax.axis_index("s")
    wid = c * 16 + s
    b = wid // _NCHUNK
    ch = wid % _NCHUNK

    pltpu.sync_copy(defo.at[b], y_v)
    pltpu.sync_copy(pts.at[b, ch], q_v)
    pltpu.sync_copy(segp.at[b, pl.ds(ch * _QC, _QC)], sp_v)
    pltpu.sync_copy(segf.at[b, pl.ds(ch * _QC, _QC)], sf_v)
    pltpu.sync_copy(prm.at[b], pm_v)

    m = [pm_v[j, :] for j in range(12)]

    zf = jnp.zeros((16,), jnp.float32)
    inf16 = jnp.full((16,), 1e30, jnp.float32)
    _QB = 8

    def qloop(g, carry):
        acc_sd, acc_kp, acc_kf = carry
        base = g * 16
        qx = q_v[0, pl.ds(base, 16)]
        qy = q_v[1, pl.ds(base, 16)]
        qz = q_v[2, pl.ds(base, 16)]
        tqx = qx * m[0] + qy * m[3] + qz * m[6] + m[9]
        tqy = qx * m[1] + qy * m[4] + qz * m[7] + m[10]
        tqz = qx * m[2] + qy * m[5] + qz * m[8] + m[11]

        for half in range(16 // _QB):
            sx = [jnp.full((16,), tqx[half * _QB + i]) for i in range(_QB)]
            sy = [jnp.full((16,), tqy[half * _QB + i]) for i in range(_QB)]
            sz = [jnp.full((16,), tqz[half * _QB + i]) for i in range(_QB)]

            def kloop(k, minvs):
                yx = y_v[0, pl.ds(k * 16, 16)]
                yy = y_v[1, pl.ds(k * 16, 16)]
                yz = y_v[2, pl.ds(k * 16, 16)]
                out = []
                for i in range(_QB):
                    dx = sx[i] - yx
                    dy = sy[i] - yy
                    dz = sz[i] - yz
                    out.append(jnp.minimum(minvs[i],
                                           dx * dx + dy * dy + dz * dz))
                return tuple(out)

            minvs = lax.fori_loop(0, _NF // 16, kloop,
                                  (inf16,) * _QB, unroll=2)
            for i in range(_QB):
                row = base + half * _QB + i
                chs = jnp.full((16,), jnp.min(minvs[i]))
                srow = sp_v[row, :]
                acc_sd = acc_sd + srow * chs
                acc_kp = acc_kp + srow
                acc_kf = acc_kf + sf_v[row, :]
        return (acc_sd, acc_kp, acc_kf)

    acc_sd, acc_kp, acc_kf = lax.fori_loop(
        0, _QC // 16, qloop, (zf, zf, zf))
    res_v[0, :] = acc_sd
    res_v[1, :] = acc_kp
    res_v[2, :] = acc_kf
    pltpu.sync_copy(res_v, out.at[b, ch])


def _tc_chamfer_kernel(pt_ref, yt_ref, segp_ref, segpr_ref, segfr_ref,
                       prm_ref, rtf_ref, rtp_ref, out_ref, acc_ref):
    b = pl.program_id(0)
    pt = pt_ref[0]
    px, py, pz = pt[0:1, :], pt[1:2, :], pt[2:3, :]
    m = [prm_ref[0, 0, j] for j in range(12)]
    tx = px * m[0] + py * m[3] + pz * m[6] + m[9]
    ty = px * m[1] + py * m[4] + pz * m[7] + m[10]
    tz = px * m[2] + py * m[5] + pz * m[8] + m[11]
    xn2 = tx * tx + ty * ty + tz * tz
    x4 = jnp.concatenate([-2.0 * tx, -2.0 * ty, -2.0 * tz,
                          jnp.ones((1, _NP), jnp.float32)], axis=0)
    yt = yt_ref[0]
    yn2 = jnp.sum(yt * yt, axis=1, keepdims=True)
    y4 = jnp.concatenate([yt, yn2], axis=1)
    d = jnp.dot(y4, x4, preferred_element_type=jnp.float32)
    cham = jnp.min(d, axis=0, keepdims=True) + xn2
    chamc = jnp.transpose(cham, (1, 0))
    segp = segp_ref[0]
    acc_ref[b, 0:1, :] = jnp.sum(segp * chamc, axis=0, keepdims=True)

    @pl.when(b == _B - 1)
    def _tail():
        seg2def = acc_ref[:, 0, :]

        def _colsum(ref):
            s = jnp.sum(ref[...], axis=1)
            acc = s[:, 0:16]
            for j in range(1, 8):
                acc = acc + s[:, 16 * j:16 * (j + 1)]
            return acc

        kp_p = _colsum(segpr_ref)
        kp_f = _colsum(segfr_ref)
        seg_def = seg2def / kp_p
        rel = jax.nn.sigmoid(seg_def / _CROSS_AVG_ERR)
        rfn = rtf_ref[...] / kp_f[..., None]
        rpn = rtp_ref[...] / kp_p[..., None]
        diff = rfn - rpn
        r_dis = jnp.sum(diff * diff, axis=-1)
        loss_rd = (r_dis - rel) ** 2
        mask = ((kp_p >= _MIN_SUPPORT) & (kp_f >= _MIN_SUPPORT)
                & (loss_rd <= _MAX_BEAR))
        maskf = mask.astype(jnp.float32)
        nofL = jnp.sum(maskf)
        total = jnp.sum(loss_rd * maskf) / (nofL + 1.0) * _CROSS_WEIGHT
        out_ref[...] = jnp.full((1, 1), jnp.where(nofL == 0.0, 0.0, total))


def _tc_tail_kernel(parts_ref, rtf_ref, rtp_ref, out_ref):
    p0 = parts_ref[:, 0, :]
    p1 = parts_ref[:, 1, :]
    p2 = parts_ref[:, 2, :]
    p3 = parts_ref[:, 3, :]
    tot = p0 + p1 + p2 + p3
    seg2def = tot[:, 0:16]
    kp_p = tot[:, 16:32]
    kp_f = tot[:, 32:48]
    seg_def = seg2def / kp_p
    rel = jax.nn.sigmoid(seg_def / _CROSS_AVG_ERR)
    rfn = rtf_ref[...] / kp_f[..., None]
    rpn = rtp_ref[...] / kp_p[..., None]
    diff = rfn - rpn
    r_dis = jnp.sum(diff * diff, axis=-1)
    loss_rd = (r_dis - rel) ** 2
    mask = ((kp_p >= _MIN_SUPPORT) & (kp_f >= _MIN_SUPPORT)
            & (loss_rd <= _MAX_BEAR))
    maskf = mask.astype(jnp.float32)
    nofL = jnp.sum(maskf)
    total = jnp.sum(loss_rd * maskf) / (nofL + 1.0) * _CROSS_WEIGHT
    out_ref[...] = jnp.full((1, 1), jnp.where(nofL == 0.0, 0.0, total))


@jax.jit
def kernel(r_tokens_full, r_tokens_partial, pc_seg_full, pc_seg_partial,
           recon_pc_full, recon_pc_partial, deformed,
           rot_full, t_full, rot_partial, t_partial,
           tgt_rand_rot, tgt_rand_t, src_rand_rot, src_rand_t):
    del recon_pc_full
    r2t = jnp.transpose(tgt_rand_rot, (0, 2, 1))
    r4t = jnp.transpose(rot_full, (0, 2, 1))
    mmat = rot_partial @ r2t @ src_rand_rot @ r4t
    cvec = ((t_partial - tgt_rand_t) @ r2t @ src_rand_rot
            + src_rand_t - t_full) @ r4t
    params = jnp.concatenate(
        [mmat.reshape(_B, 9), cvec.reshape(_B, 3)], axis=1)
    params = params.reshape(_B, 1, 12)
    pt = jnp.transpose(recon_pc_partial, (0, 2, 1))
    yt = jnp.transpose(deformed, (0, 2, 1))

    out = pl.pallas_call(
        _tc_chamfer_kernel,
        grid=(_B,),
        in_specs=[
            pl.BlockSpec((1, 3, _NP), lambda b: (b, 0, 0)),
            pl.BlockSpec((1, _NF, 3), lambda b: (b, 0, 0)),
            pl.BlockSpec((1, _NP, _KP), lambda b: (b, 0, 0)),
            pl.BlockSpec((_B, _NP * _KP // 128, 128), lambda b: (0, 0, 0)),
            pl.BlockSpec((_B, _NF * _KP // 128, 128), lambda b: (0, 0, 0)),
            pl.BlockSpec((1, 1, 12), lambda b: (b, 0, 0),
                         memory_space=pltpu.SMEM),
            pl.BlockSpec((_B, _KP, _D), lambda b: (0, 0, 0)),
            pl.BlockSpec((_B, _KP, _D), lambda b: (0, 0, 0)),
        ],
        out_specs=pl.BlockSpec((1, 1), lambda b: (0, 0)),
        out_shape=jax.ShapeDtypeStruct((1, 1), jnp.float32),
        scratch_shapes=[pltpu.VMEM((_B, 1, _KP), jnp.float32)],
        compiler_params=pltpu.CompilerParams(
            dimension_semantics=("arbitrary",)),
    )(pt, yt, pc_seg_partial,
      pc_seg_partial.reshape(_B, _NP * _KP // 128, 128),
      pc_seg_full.reshape(_B, _NF * _KP // 128, 128),
      params, r_tokens_full, r_tokens_partial)
    return out[0, 0]


# speedup vs baseline: 1.2808x; 1.2808x over previous
import functools

import jax
import jax.numpy as jnp
from jax import lax
from jax.experimental import pallas as pl
from jax.experimental.pallas import tpu as pltpu
from jax.experimental.pallas import tpu_sc as plsc

_CROSS_AVG_ERR = 0.25
_CROSS_WEIGHT = 1.0
_MIN_SUPPORT = 20.0
_MAX_BEAR = 20.0

_B = 8
_NP = 2048
_NF = 2048
_KP = 16
_D = 128
_NCHUNK = 4
_QC = _NP // _NCHUNK


def _sc_chamfer_kernel(pts, defo, segp, segf, prm, out,
                       y_v, q_v, sp_v, sf_v, pm_v, res_v):
    c = lax.axis_index("c")
    s = lax.axis_index("s")
    wid = c * 16 + s
    b = wid // _NCHUNK
    ch = wid % _NCHUNK

    pltpu.sync_copy(defo.at[b], y_v)
    pltpu.sync_copy(pts.at[b, ch], q_v)
    pltpu.sync_copy(segp.at[b, pl.ds(ch * _QC, _QC)], sp_v)
    pltpu.sync_copy(segf.at[b, pl.ds(ch * _QC, _QC)], sf_v)
    pltpu.sync_copy(prm.at[b], pm_v)

    m = [pm_v[j, :] for j in range(12)]

    zf = jnp.zeros((16,), jnp.float32)
    inf16 = jnp.full((16,), 1e30, jnp.float32)
    _QB = 8

    def qloop(g, carry):
        acc_sd, acc_kp, acc_kf = carry
        base = g * 16
        qx = q_v[0, pl.ds(base, 16)]
        qy = q_v[1, pl.ds(base, 16)]
        qz = q_v[2, pl.ds(base, 16)]
        tqx = qx * m[0] + qy * m[3] + qz * m[6] + m[9]
        tqy = qx * m[1] + qy * m[4] + qz * m[7] + m[10]
        tqz = qx * m[2] + qy * m[5] + qz * m[8] + m[11]

        for half in range(16 // _QB):
            sx = [jnp.full((16,), tqx[half * _QB + i]) for i in range(_QB)]
            sy = [jnp.full((16,), tqy[half * _QB + i]) for i in range(_QB)]
            sz = [jnp.full((16,), tqz[half * _QB + i]) for i in range(_QB)]

            def kloop(k, minvs):
                yx = y_v[0, pl.ds(k * 16, 16)]
                yy = y_v[1, pl.ds(k * 16, 16)]
                yz = y_v[2, pl.ds(k * 16, 16)]
                out = []
                for i in range(_QB):
                    dx = sx[i] - yx
                    dy = sy[i] - yy
                    dz = sz[i] - yz
                    out.append(jnp.minimum(minvs[i],
                                           dx * dx + dy * dy + dz * dz))
                return tuple(out)

            minvs = lax.fori_loop(0, _NF // 16, kloop,
                                  (inf16,) * _QB, unroll=2)
            for i in range(_QB):
                row = base + half * _QB + i
                chs = jnp.full((16,), jnp.min(minvs[i]))
                srow = sp_v[row, :]
                acc_sd = acc_sd + srow * chs
                acc_kp = acc_kp + srow
                acc_kf = acc_kf + sf_v[row, :]
        return (acc_sd, acc_kp, acc_kf)

    acc_sd, acc_kp, acc_kf = lax.fori_loop(
        0, _QC // 16, qloop, (zf, zf, zf))
    res_v[0, :] = acc_sd
    res_v[1, :] = acc_kp
    res_v[2, :] = acc_kf
    pltpu.sync_copy(res_v, out.at[b, ch])


def _tc_chamfer_kernel(pt_ref, yt_ref, segp_ref, segf_ref,
                       prm_ref, rtf_ref, rtp_ref, out_ref, acc_ref):
    b = pl.program_id(0)
    pt = pt_ref[0]
    px, py, pz = pt[0:1, :], pt[1:2, :], pt[2:3, :]
    m = [prm_ref[0, 0, j] for j in range(12)]
    tx = px * m[0] + py * m[3] + pz * m[6] + m[9]
    ty = px * m[1] + py * m[4] + pz * m[7] + m[10]
    tz = px * m[2] + py * m[5] + pz * m[8] + m[11]
    xn2 = tx * tx + ty * ty + tz * tz
    x4 = jnp.concatenate([-2.0 * tx, -2.0 * ty, -2.0 * tz,
                          jnp.ones((1, _NP), jnp.float32)], axis=0)
    yt = yt_ref[0]
    yn2 = jnp.sum(yt * yt, axis=1, keepdims=True)
    y4 = jnp.concatenate([yt, yn2], axis=1)
    d = jnp.dot(y4, x4, preferred_element_type=jnp.float32)
    cham = jnp.min(d, axis=0, keepdims=True) + xn2
    chamc = jnp.transpose(cham, (1, 0))
    segp = segp_ref[0]
    acc_ref[b, 0:1, :] = jnp.sum(segp * chamc, axis=0, keepdims=True)
    acc_ref[b, 1:2, :] = jnp.sum(segp, axis=0, keepdims=True)
    acc_ref[b, 2:3, :] = jnp.sum(segf_ref[0], axis=0, keepdims=True)

    @pl.when(b == _B - 1)
    def _tail():
        seg2def = acc_ref[:, 0, :]
        kp_p = acc_ref[:, 1, :]
        kp_f = acc_ref[:, 2, :]
        seg_def = seg2def / kp_p
        rel = jax.nn.sigmoid(seg_def / _CROSS_AVG_ERR)
        rfn = rtf_ref[...] / kp_f[..., None]
        rpn = rtp_ref[...] / kp_p[..., None]
        diff = rfn - rpn
        r_dis = jnp.sum(diff * diff, axis=-1)
        loss_rd = (r_dis - rel) ** 2
        mask = ((kp_p >= _MIN_SUPPORT) & (kp_f >= _MIN_SUPPORT)
                & (loss_rd <= _MAX_BEAR))
        maskf = mask.astype(jnp.float32)
        nofL = jnp.sum(maskf)
        total = jnp.sum(loss_rd * maskf) / (nofL + 1.0) * _CROSS_WEIGHT
        out_ref[...] = jnp.full((1, 1), jnp.where(nofL == 0.0, 0.0, total))


def _tc_tail_kernel(parts_ref, rtf_ref, rtp_ref, out_ref):
    p0 = parts_ref[:, 0, :]
    p1 = parts_ref[:, 1, :]
    p2 = parts_ref[:, 2, :]
    p3 = parts_ref[:, 3, :]
    tot = p0 + p1 + p2 + p3
    seg2def = tot[:, 0:16]
    kp_p = tot[:, 16:32]
    kp_f = tot[:, 32:48]
    seg_def = seg2def / kp_p
    rel = jax.nn.sigmoid(seg_def / _CROSS_AVG_ERR)
    rfn = rtf_ref[...] / kp_f[..., None]
    rpn = rtp_ref[...] / kp_p[..., None]
    diff = rfn - rpn
    r_dis = jnp.sum(diff * diff, axis=-1)
    loss_rd = (r_dis - rel) ** 2
    mask = ((kp_p >= _MIN_SUPPORT) & (kp_f >= _MIN_SUPPORT)
            & (loss_rd <= _MAX_BEAR))
    maskf = mask.astype(jnp.float32)
    nofL = jnp.sum(maskf)
    total = jnp.sum(loss_rd * maskf) / (nofL + 1.0) * _CROSS_WEIGHT
    out_ref[...] = jnp.full((1, 1), jnp.where(nofL == 0.0, 0.0, total))


@jax.jit
def kernel(r_tokens_full, r_tokens_partial, pc_seg_full, pc_seg_partial,
           recon_pc_full, recon_pc_partial, deformed,
           rot_full, t_full, rot_partial, t_partial,
           tgt_rand_rot, tgt_rand_t, src_rand_rot, src_rand_t):
    del recon_pc_full
    r2t = jnp.transpose(tgt_rand_rot, (0, 2, 1))
    r4t = jnp.transpose(rot_full, (0, 2, 1))
    mmat = rot_partial @ r2t @ src_rand_rot @ r4t
    cvec = ((t_partial - tgt_rand_t) @ r2t @ src_rand_rot
            + src_rand_t - t_full) @ r4t
    params = jnp.concatenate(
        [mmat.reshape(_B, 9), cvec.reshape(_B, 3)], axis=1)
    params = params.reshape(_B, 1, 12)
    pt = jnp.transpose(recon_pc_partial, (0, 2, 1))
    yt = jnp.transpose(deformed, (0, 2, 1))

    out = pl.pallas_call(
        _tc_chamfer_kernel,
        grid=(_B,),
        in_specs=[
            pl.BlockSpec((1, 3, _NP), lambda b: (b, 0, 0)),
            pl.BlockSpec((1, _NF, 3), lambda b: (b, 0, 0)),
            pl.BlockSpec((1, _NP, _KP), lambda b: (b, 0, 0)),
            pl.BlockSpec((1, _NF, _KP), lambda b: (b, 0, 0)),
            pl.BlockSpec((1, 1, 12), lambda b: (b, 0, 0),
                         memory_space=pltpu.SMEM),
            pl.BlockSpec((_B, _KP, _D), lambda b: (0, 0, 0)),
            pl.BlockSpec((_B, _KP, _D), lambda b: (0, 0, 0)),
        ],
        out_specs=pl.BlockSpec((1, 1), lambda b: (0, 0)),
        out_shape=jax.ShapeDtypeStruct((1, 1), jnp.float32),
        scratch_shapes=[pltpu.VMEM((_B, 3, _KP), jnp.float32)],
        compiler_params=pltpu.CompilerParams(
            dimension_semantics=("arbitrary",)),
    )(pt, yt, pc_seg_partial, pc_seg_full,
      params, r_tokens_full, r_tokens_partial)
    return out[0, 0]


# speedup vs baseline: 1.3796x; 1.0771x over previous
import functools

import jax
import jax.numpy as jnp
from jax import lax
from jax.experimental import pallas as pl
from jax.experimental.pallas import tpu as pltpu
from jax.experimental.pallas import tpu_sc as plsc

_CROSS_AVG_ERR = 0.25
_CROSS_WEIGHT = 1.0
_MIN_SUPPORT = 20.0
_MAX_BEAR = 20.0

_B = 8
_NP = 2048
_NF = 2048
_KP = 16
_D = 128
_NCHUNK = 4
_QC = _NP // _NCHUNK


def _sc_chamfer_kernel(pts, defo, segp, segf, prm, out,
                       y_v, q_v, sp_v, sf_v, pm_v, res_v):
    c = lax.axis_index("c")
    s = lax.axis_index("s")
    wid = c * 16 + s
    b = wid // _NCHUNK
    ch = wid % _NCHUNK

    pltpu.sync_copy(defo.at[b], y_v)
    pltpu.sync_copy(pts.at[b, ch], q_v)
    pltpu.sync_copy(segp.at[b, pl.ds(ch * _QC, _QC)], sp_v)
    pltpu.sync_copy(segf.at[b, pl.ds(ch * _QC, _QC)], sf_v)
    pltpu.sync_copy(prm.at[b], pm_v)

    m = [pm_v[j, :] for j in range(12)]

    zf = jnp.zeros((16,), jnp.float32)
    inf16 = jnp.full((16,), 1e30, jnp.float32)
    _QB = 8

    def qloop(g, carry):
        acc_sd, acc_kp, acc_kf = carry
        base = g * 16
        qx = q_v[0, pl.ds(base, 16)]
        qy = q_v[1, pl.ds(base, 16)]
        qz = q_v[2, pl.ds(base, 16)]
        tqx = qx * m[0] + qy * m[3] + qz * m[6] + m[9]
        tqy = qx * m[1] + qy * m[4] + qz * m[7] + m[10]
        tqz = qx * m[2] + qy * m[5] + qz * m[8] + m[11]

        for half in range(16 // _QB):
            sx = [jnp.full((16,), tqx[half * _QB + i]) for i in range(_QB)]
            sy = [jnp.full((16,), tqy[half * _QB + i]) for i in range(_QB)]
            sz = [jnp.full((16,), tqz[half * _QB + i]) for i in range(_QB)]

            def kloop(k, minvs):
                yx = y_v[0, pl.ds(k * 16, 16)]
                yy = y_v[1, pl.ds(k * 16, 16)]
                yz = y_v[2, pl.ds(k * 16, 16)]
                out = []
                for i in range(_QB):
                    dx = sx[i] - yx
                    dy = sy[i] - yy
                    dz = sz[i] - yz
                    out.append(jnp.minimum(minvs[i],
                                           dx * dx + dy * dy + dz * dz))
                return tuple(out)

            minvs = lax.fori_loop(0, _NF // 16, kloop,
                                  (inf16,) * _QB, unroll=2)
            for i in range(_QB):
                row = base + half * _QB + i
                chs = jnp.full((16,), jnp.min(minvs[i]))
                srow = sp_v[row, :]
                acc_sd = acc_sd + srow * chs
                acc_kp = acc_kp + srow
                acc_kf = acc_kf + sf_v[row, :]
        return (acc_sd, acc_kp, acc_kf)

    acc_sd, acc_kp, acc_kf = lax.fori_loop(
        0, _QC // 16, qloop, (zf, zf, zf))
    res_v[0, :] = acc_sd
    res_v[1, :] = acc_kp
    res_v[2, :] = acc_kf
    pltpu.sync_copy(res_v, out.at[b, ch])


def _tc_chamfer_kernel(pt_ref, yt_ref, segp_ref, segf_ref,
                       prm_ref, rtf_ref, rtp_ref, out_ref, acc_ref):
    b = pl.program_id(0)
    pt = pt_ref[0]
    px, py, pz = pt[0:1, :], pt[1:2, :], pt[2:3, :]
    m = [prm_ref[0, 0, j] for j in range(12)]
    tx = px * m[0] + py * m[3] + pz * m[6] + m[9]
    ty = px * m[1] + py * m[4] + pz * m[7] + m[10]
    tz = px * m[2] + py * m[5] + pz * m[8] + m[11]
    xn2 = tx * tx + ty * ty + tz * tz
    x4 = jnp.concatenate([-2.0 * tx, -2.0 * ty, -2.0 * tz,
                          jnp.ones((1, _NP), jnp.float32)], axis=0)
    yr = yt_ref[0]
    yn2 = (yr[0:1, :] * yr[0:1, :] + yr[1:2, :] * yr[1:2, :]
           + yr[2:3, :] * yr[2:3, :])
    y4t = jnp.concatenate([yr, yn2], axis=0)
    d = jax.lax.dot_general(
        y4t, x4, dimension_numbers=(((0,), (0,)), ((), ())),
        preferred_element_type=jnp.float32)
    cham = jnp.min(d, axis=0, keepdims=True) + xn2
    chamc = jnp.transpose(cham, (1, 0))
    segp = segp_ref[0]
    acc_ref[b, 0:1, :] = jnp.sum(segp * chamc, axis=0, keepdims=True)
    acc_ref[b, 1:2, :] = jnp.sum(segp, axis=0, keepdims=True)
    acc_ref[b, 2:3, :] = jnp.sum(segf_ref[0], axis=0, keepdims=True)

    @pl.when(b == _B - 1)
    def _tail():
        seg2def = acc_ref[:, 0, :]
        kp_p = acc_ref[:, 1, :]
        kp_f = acc_ref[:, 2, :]
        seg_def = seg2def / kp_p
        rel = jax.nn.sigmoid(seg_def / _CROSS_AVG_ERR)
        rfn = rtf_ref[...] / kp_f[..., None]
        rpn = rtp_ref[...] / kp_p[..., None]
        diff = rfn - rpn
        r_dis = jnp.sum(diff * diff, axis=-1)
        loss_rd = (r_dis - rel) ** 2
        mask = ((kp_p >= _MIN_SUPPORT) & (kp_f >= _MIN_SUPPORT)
                & (loss_rd <= _MAX_BEAR))
        maskf = mask.astype(jnp.float32)
        nofL = jnp.sum(maskf)
        total = jnp.sum(loss_rd * maskf) / (nofL + 1.0) * _CROSS_WEIGHT
        out_ref[...] = jnp.full((1, 1), jnp.where(nofL == 0.0, 0.0, total))


def _tc_tail_kernel(parts_ref, rtf_ref, rtp_ref, out_ref):
    p0 = parts_ref[:, 0, :]
    p1 = parts_ref[:, 1, :]
    p2 = parts_ref[:, 2, :]
    p3 = parts_ref[:, 3, :]
    tot = p0 + p1 + p2 + p3
    seg2def = tot[:, 0:16]
    kp_p = tot[:, 16:32]
    kp_f = tot[:, 32:48]
    seg_def = seg2def / kp_p
    rel = jax.nn.sigmoid(seg_def / _CROSS_AVG_ERR)
    rfn = rtf_ref[...] / kp_f[..., None]
    rpn = rtp_ref[...] / kp_p[..., None]
    diff = rfn - rpn
    r_dis = jnp.sum(diff * diff, axis=-1)
    loss_rd = (r_dis - rel) ** 2
    mask = ((kp_p >= _MIN_SUPPORT) & (kp_f >= _MIN_SUPPORT)
            & (loss_rd <= _MAX_BEAR))
    maskf = mask.astype(jnp.float32)
    nofL = jnp.sum(maskf)
    total = jnp.sum(loss_rd * maskf) / (nofL + 1.0) * _CROSS_WEIGHT
    out_ref[...] = jnp.full((1, 1), jnp.where(nofL == 0.0, 0.0, total))


@jax.jit
def kernel(r_tokens_full, r_tokens_partial, pc_seg_full, pc_seg_partial,
           recon_pc_full, recon_pc_partial, deformed,
           rot_full, t_full, rot_partial, t_partial,
           tgt_rand_rot, tgt_rand_t, src_rand_rot, src_rand_t):
    del recon_pc_full
    r2t = jnp.transpose(tgt_rand_rot, (0, 2, 1))
    r4t = jnp.transpose(rot_full, (0, 2, 1))
    mmat = rot_partial @ r2t @ src_rand_rot @ r4t
    cvec = ((t_partial - tgt_rand_t) @ r2t @ src_rand_rot
            + src_rand_t - t_full) @ r4t
    params = jnp.concatenate(
        [mmat.reshape(_B, 9), cvec.reshape(_B, 3)], axis=1)
    params = params.reshape(_B, 1, 12)
    pt = jnp.transpose(recon_pc_partial, (0, 2, 1))

    out = pl.pallas_call(
        _tc_chamfer_kernel,
        grid=(_B,),
        in_specs=[
            pl.BlockSpec((1, 3, _NP), lambda b: (b, 0, 0)),
            pl.BlockSpec((1, 3, _NF), lambda b: (b, 0, 0)),
            pl.BlockSpec((1, _NP, _KP), lambda b: (b, 0, 0)),
            pl.BlockSpec((1, _NF, _KP), lambda b: (b, 0, 0)),
            pl.BlockSpec((1, 1, 12), lambda b: (b, 0, 0),
                         memory_space=pltpu.SMEM),
            pl.BlockSpec((_B, _KP, _D), lambda b: (0, 0, 0)),
            pl.BlockSpec((_B, _KP, _D), lambda b: (0, 0, 0)),
        ],
        out_specs=pl.BlockSpec((1, 1), lambda b: (0, 0)),
        out_shape=jax.ShapeDtypeStruct((1, 1), jnp.float32),
        scratch_shapes=[pltpu.VMEM((_B, 3, _KP), jnp.float32)],
        compiler_params=pltpu.CompilerParams(
            dimension_semantics=("arbitrary",)),
    )(pt, deformed, pc_seg_partial, pc_seg_full,
      params, r_tokens_full, r_tokens_partial)
    return out[0, 0]
